# baseline (speedup 1.0000x reference)
import functools

import jax
import jax.numpy as jnp
from jax import lax
from jax.experimental import pallas as pl
from jax.experimental.pallas import tpu as pltpu
from jax.experimental.pallas import tpu_sc as plsc

NC = 2
NS = 16
NW = NC * NS
CH = 128
GPT = 80
EPT = CH * GPT
EP = NW * EPT
NP = 10240
STRIPE = NP // NS
D = 128
ROWB = 512
GRID = NP // ROWB

_mesh = plsc.VectorSubcoreMesh(
    core_axis_name="c", subcore_axis_name="s", num_cores=NC, num_subcores=NS
)


def _count_body(dst_ref, out_ref, idx_v, cnt_v):
    cid = lax.axis_index("c")
    sid = lax.axis_index("s")
    wid = cid * NS + sid
    pltpu.sync_copy(dst_ref.at[wid], idx_v)
    zeros = jnp.zeros((16,), jnp.float32)

    def zero_body(i, carry):
        cnt_v[pl.ds(i * 16, 16)] = zeros
        return carry

    lax.fori_loop(0, NP // 16, zero_body, 0)
    ones = jnp.full((16,), 1.0, jnp.float32)

    def scat_body(i, carry):
        idx = idx_v[pl.ds(i * 16, 16)]
        plsc.addupdate_scatter(cnt_v, [idx], ones)
        return carry

    lax.fori_loop(0, EPT // 16, scat_body, 0)
    pltpu.sync_copy(cnt_v, out_ref.at[wid])


_count_kernel = functools.partial(
    pl.kernel,
    out_type=jax.ShapeDtypeStruct((NW, NP), jnp.float32),
    mesh=_mesh,
    compiler_params=pltpu.CompilerParams(needs_layout_passes=False),
    scratch_types=[
        pltpu.VMEM((EPT,), jnp.int32),
        pltpu.VMEM((NP,), jnp.float32),
    ],
)(_count_body)


def _mm_body(x_ref, w_ref, cnt_ref, h2_ref):
    cnt = jnp.sum(cnt_ref[...], axis=0)
    dinv = lax.rsqrt(cnt + 1.0)
    h = jnp.dot(x_ref[...], w_ref[...], preferred_element_type=jnp.float32)
    h2_ref[...] = h * dinv[:, None]


SCH = 64
BLK = 8
NB = 4
ROWS0 = 96
ROWS1 = 64
NBLK0 = ROWS0 * CH // (BLK * SCH)
NBLK1 = ROWS1 * CH // (BLK * SCH)
PADR = (max(NBLK0, NBLK1) + 2) * BLK * SCH // CH
E0 = NS * ROWS0 * CH


def _agg_body(h2_ref, srcg_ref, dstg_ref, out_ref,
              acc_sh, idxs_v, idxd0, idxd1, buf0, buf1, buf2, buf3,
              sg0, sg1, sg2, sg3, ss0, ss1, ss2, ss3, si0, si1):
    cid = lax.axis_index("c")
    sid = lax.axis_index("s")
    wid = cid * NS + sid
    pltpu.sync_copy(srcg_ref.at[wid], idxs_v)

    zeros = jnp.zeros((16,), jnp.float32)

    def zero_body(r, carry):
        for c in range(D // 16):
            buf0[r, pl.ds(c * 16, 16)] = zeros
        return carry

    lax.fori_loop(0, SCH, zero_body, 0)
    base = sid * STRIPE
    for z in range(STRIPE // SCH):
        pltpu.sync_copy(buf0, acc_sh.at[pl.ds(base + z * SCH, SCH)])
    plsc.subcore_barrier()

    bufs = (buf0, buf1, buf2, buf3)
    sgs = (sg0, sg1, sg2, sg3)
    sss = (ss0, ss1, ss2, ss3)
    idxds = (idxd0, idxd1)
    sis = (si0, si1)

    def gather(row, half, bi):
        return pltpu.make_async_copy(
            h2_ref.at[idxs_v.at[row, pl.ds(half * SCH, SCH)]],
            bufs[bi], sgs[bi])

    def scat_wait(bi, sub):
        pltpu.make_async_copy(bufs[bi], acc_sh.at[idxds[sub].at[0]],
                              sss[bi]).wait()

    def iblock(g, sub):
        return pltpu.make_async_copy(dstg_ref.at[wid, g], idxds[sub],
                                     sis[sub])

    def span(g0, nblocks):
        iblock(g0, 0).start()
        iblock(g0 + 1, 1).start()
        gather(g0 * (BLK // 2), 0, 0).start()
        gather(g0 * (BLK // 2), 1, 1).start()

        def loop(gp, carry):
            for sub in (0, 1):
                g = g0 + gp * 2 + sub
                row0 = g * (BLK // 2)
                krel0 = (gp * 2 + sub) * BLK
                iblock(g, sub).wait()
                for t in range(BLK):
                    bi = t % NB
                    gather(row0 + t // 2, t % 2, bi).wait()
                    pltpu.async_copy(bufs[bi], acc_sh.at[idxds[sub].at[t]],
                                     sss[bi], add=True)
                    bj = (t + 2) % NB

                    @pl.when(krel0 + t >= 2)
                    def _():
                        scat_wait(bj, sub)

                    gather(row0 + (t + 2) // 2, t % 2, bj).start()
                iblock(g + 2, sub).start()
            return carry

        lax.fori_loop(0, nblocks // 2, loop, 0)
        gend = g0 + nblocks
        scat_wait(2, 0)
        scat_wait(3, 0)
        gather(gend * (BLK // 2), 0, 0).wait()
        gather(gend * (BLK // 2), 1, 1).wait()
        iblock(gend, 0).wait()
        iblock(gend + 1, 1).wait()

    span(0, NBLK1)

    @pl.when(cid == 0)
    def _():
        span(NBLK1, NBLK0 - NBLK1)

    plsc.subcore_barrier()

    NZ = STRIPE // SCH

    def cin(z, b):
        return pltpu.make_async_copy(
            acc_sh.at[pl.ds(base + z * SCH, SCH)], bufs[b], sgs[b])

    def cout(z, b):
        return pltpu.make_async_copy(
            bufs[b], out_ref.at[cid, pl.ds(base + z * SCH, SCH)], sss[b])

    for z in range(NB):
        cin(z, z).start()
    for z in range(NZ):
        b = z % NB
        cin(z, b).wait()
        cout(z, b).start()
        if z + NB < NZ:
            cout(z, b).wait()
            cin(z + NB, b).start()
    for z in range(NZ - NB, NZ):
        cout(z, z % NB).wait()


_agg_kernel = functools.partial(
    pl.kernel,
    out_type=jax.ShapeDtypeStruct((NC, NP, D), jnp.float32),
    mesh=_mesh,
    scratch_types=[
        pltpu.VMEM_SHARED((NP, D), jnp.float32),
        pltpu.VMEM((PADR, CH), jnp.int32),
        pltpu.VMEM((BLK, SCH), jnp.int32),
        pltpu.VMEM((BLK, SCH), jnp.int32),
        pltpu.VMEM((SCH, D), jnp.float32),
        pltpu.VMEM((SCH, D), jnp.float32),
        pltpu.VMEM((SCH, D), jnp.float32),
        pltpu.VMEM((SCH, D), jnp.float32),
        pltpu.SemaphoreType.DMA,
        pltpu.SemaphoreType.DMA,
        pltpu.SemaphoreType.DMA,
        pltpu.SemaphoreType.DMA,
        pltpu.SemaphoreType.DMA,
        pltpu.SemaphoreType.DMA,
        pltpu.SemaphoreType.DMA,
        pltpu.SemaphoreType.DMA,
        pltpu.SemaphoreType.DMA,
        pltpu.SemaphoreType.DMA,
    ],
)(_agg_body)


def _fin_body(acc_ref, h2_ref, cnt_ref, b_ref, o_ref):
    cnt = jnp.sum(cnt_ref[...], axis=0)
    dinv = lax.rsqrt(cnt + 1.0)[:, None]
    s = acc_ref[0] + acc_ref[1] + h2_ref[...]
    o_ref[...] = jnp.maximum(s * dinv + b_ref[...], 0.0)


def kernel(x, edge_index, W, b):
    N = x.shape[0]
    E = edge_index.shape[1]
    ei = edge_index.astype(jnp.int32)
    pad = jnp.full((EP - E,), NP - 1, jnp.int32)
    src = jnp.concatenate([ei[0], pad])
    dst = jnp.concatenate([ei[1], pad])
    def tile_layout(a):
        a0 = a[:E0].reshape(NS, ROWS0, CH)
        a1 = a[E0:].reshape(NS, ROWS1, CH)
        p0 = jnp.full((NS, PADR - ROWS0, CH), NP - 1, jnp.int32)
        p1 = jnp.full((NS, PADR - ROWS1, CH), NP - 1, jnp.int32)
        return jnp.concatenate([
            jnp.concatenate([a0, p0], axis=1),
            jnp.concatenate([a1, p1], axis=1)], axis=0)

    srcg = tile_layout(src)
    dstg = tile_layout(dst).reshape(NW, max(NBLK0, NBLK1) + 2, BLK, SCH)
    dst_flat = dst.reshape(NW, EPT)
    x_p = jnp.pad(x, ((0, NP - N), (0, 0)))

    cnt_part = _count_kernel(dst_flat)

    h2 = pl.pallas_call(
        _mm_body,
        grid=(GRID,),
        in_specs=[
            pl.BlockSpec((ROWB, D), lambda i: (i, 0)),
            pl.BlockSpec((D, D), lambda i: (0, 0)),
            pl.BlockSpec((NW, ROWB), lambda i: (0, i)),
        ],
        out_specs=pl.BlockSpec((ROWB, D), lambda i: (i, 0)),
        out_shape=jax.ShapeDtypeStruct((NP, D), jnp.float32),
    )(x_p, W, cnt_part)

    accp = _agg_kernel(h2, srcg, dstg)

    out_p = pl.pallas_call(
        _fin_body,
        grid=(GRID,),
        in_specs=[
            pl.BlockSpec((NC, ROWB, D), lambda i: (0, i, 0)),
            pl.BlockSpec((ROWB, D), lambda i: (i, 0)),
            pl.BlockSpec((NW, ROWB), lambda i: (0, i)),
            pl.BlockSpec((1, D), lambda i: (0, 0)),
        ],
        out_specs=pl.BlockSpec((ROWB, D), lambda i: (i, 0)),
        out_shape=jax.ShapeDtypeStruct((NP, D), jnp.float32),
    )(accp, h2, cnt_part, b.reshape(1, D))

    return out_p[:N]


# speedup vs baseline: 1.6454x; 1.6454x over previous
import functools

import jax
import jax.numpy as jnp
from jax import lax
from jax.experimental import pallas as pl
from jax.experimental.pallas import tpu as pltpu
from jax.experimental.pallas import tpu_sc as plsc

NC = 2
NS = 16
NW = NC * NS
CH = 128
GPT = 80
EPT = CH * GPT
EP = NW * EPT
NP = 10240
STRIPE = NP // NS
D = 128
ROWB = 512
GRID = NP // ROWB

_mesh = plsc.VectorSubcoreMesh(
    core_axis_name="c", subcore_axis_name="s", num_cores=NC, num_subcores=NS
)


def _count_body(dst_ref, out_ref, idx_v, cnt_v):
    cid = lax.axis_index("c")
    sid = lax.axis_index("s")
    wid = cid * NS + sid
    pltpu.sync_copy(dst_ref.at[wid], idx_v)
    zeros = jnp.zeros((16,), jnp.float32)

    def zero_body(i, carry):
        cnt_v[pl.ds(i * 16, 16)] = zeros
        return carry

    lax.fori_loop(0, NP // 16, zero_body, 0)
    ones = jnp.full((16,), 1.0, jnp.float32)

    def scat_body(i, carry):
        idx = idx_v[pl.ds(i * 16, 16)]
        plsc.addupdate_scatter(cnt_v, [idx], ones)
        return carry

    lax.fori_loop(0, EPT // 16, scat_body, 0)
    pltpu.sync_copy(cnt_v, out_ref.at[wid])


_count_kernel = functools.partial(
    pl.kernel,
    out_type=jax.ShapeDtypeStruct((NW, NP), jnp.float32),
    mesh=_mesh,
    compiler_params=pltpu.CompilerParams(needs_layout_passes=False),
    scratch_types=[
        pltpu.VMEM((EPT,), jnp.int32),
        pltpu.VMEM((NP,), jnp.float32),
    ],
)(_count_body)


def _mm_body(x_ref, w_ref, cnt_ref, h2_ref):
    cnt = jnp.sum(cnt_ref[...], axis=0)
    dinv = lax.rsqrt(cnt + 1.0)
    h = jnp.dot(x_ref[...], w_ref[...], preferred_element_type=jnp.float32)
    h2_ref[...] = h * dinv[:, None]


SCH = 64
NCHK = EPT // SCH
BLK = 8
NBLK = NCHK // BLK
NB = 4


def _agg_body(h2_ref, srcg_ref, dstg_ref, out_ref,
              acc_sh, idxs_v, idxd0, idxd1, buf0, buf1, buf2, buf3,
              sg0, sg1, sg2, sg3, ss0, ss1, ss2, ss3, si0, si1):
    cid = lax.axis_index("c")
    sid = lax.axis_index("s")
    wid = cid * NS + sid
    pltpu.sync_copy(srcg_ref.at[wid], idxs_v)

    zeros = jnp.zeros((16,), jnp.float32)

    def zero_body(r, carry):
        for c in range(D // 16):
            buf0[r, pl.ds(c * 16, 16)] = zeros
        return carry

    lax.fori_loop(0, SCH, zero_body, 0)
    base = sid * STRIPE
    for z in range(STRIPE // SCH):
        pltpu.sync_copy(buf0, acc_sh.at[pl.ds(base + z * SCH, SCH)])
    plsc.subcore_barrier()

    bufs = (buf0, buf1, buf2, buf3)
    sgs = (sg0, sg1, sg2, sg3)
    sss = (ss0, ss1, ss2, ss3)
    idxds = (idxd0, idxd1)
    sis = (si0, si1)

    def gather(row, half, bi):
        return pltpu.make_async_copy(
            h2_ref.at[idxs_v.at[row, pl.ds(half * SCH, SCH)]],
            bufs[bi], sgs[bi])

    def scat_wait(bi, sub):
        pltpu.make_async_copy(bufs[bi], acc_sh.at[idxds[sub].at[0]],
                              sss[bi]).wait()

    def iblock(g, sub):
        return pltpu.make_async_copy(dstg_ref.at[wid, g], idxds[sub],
                                     sis[sub])

    iblock(0, 0).start()
    iblock(1, 1).start()
    gather(0, 0, 0).start()
    gather(0, 1, 1).start()

    def loop(gp, carry):
        for sub in (0, 1):
            g = gp * 2 + sub
            row0 = g * (BLK // 2)
            iblock(g, sub).wait()
            for t in range(BLK):
                bi = t % NB
                k = g * BLK + t
                gather(row0 + t // 2, t % 2, bi).wait()
                pltpu.async_copy(bufs[bi], acc_sh.at[idxds[sub].at[t]],
                                 sss[bi], add=True)
                bj = (t + 2) % NB

                @pl.when(k >= 2)
                def _():
                    scat_wait(bj, sub)

                gather(row0 + (t + 2) // 2, t % 2, bj).start()
            iblock(g + 2, sub).start()
        return carry

    lax.fori_loop(0, NBLK // 2, loop, 0)
    scat_wait(2, 0)
    scat_wait(3, 0)
    gather(NCHK // 2, 0, 0).wait()
    gather(NCHK // 2, 1, 1).wait()
    iblock(NBLK, 0).wait()
    iblock(NBLK + 1, 1).wait()
    plsc.subcore_barrier()

    NZ = STRIPE // SCH

    def cin(z, b):
        return pltpu.make_async_copy(
            acc_sh.at[pl.ds(base + z * SCH, SCH)], bufs[b], sgs[b])

    def cout(z, b):
        return pltpu.make_async_copy(
            bufs[b], out_ref.at[cid, pl.ds(base + z * SCH, SCH)], sss[b])

    for z in range(NB):
        cin(z, z).start()
    for z in range(NZ):
        b = z % NB
        cin(z, b).wait()
        cout(z, b).start()
        if z + NB < NZ:
            cout(z, b).wait()
            cin(z + NB, b).start()
    for z in range(NZ - NB, NZ):
        cout(z, z % NB).wait()


_agg_kernel = functools.partial(
    pl.kernel,
    out_type=jax.ShapeDtypeStruct((NC, NP, D), jnp.float32),
    mesh=_mesh,
    scratch_types=[
        pltpu.VMEM_SHARED((NP, D), jnp.float32),
        pltpu.VMEM((GPT + 2, CH), jnp.int32),
        pltpu.VMEM((BLK, SCH), jnp.int32),
        pltpu.VMEM((BLK, SCH), jnp.int32),
        pltpu.VMEM((SCH, D), jnp.float32),
        pltpu.VMEM((SCH, D), jnp.float32),
        pltpu.VMEM((SCH, D), jnp.float32),
        pltpu.VMEM((SCH, D), jnp.float32),
        pltpu.SemaphoreType.DMA,
        pltpu.SemaphoreType.DMA,
        pltpu.SemaphoreType.DMA,
        pltpu.SemaphoreType.DMA,
        pltpu.SemaphoreType.DMA,
        pltpu.SemaphoreType.DMA,
        pltpu.SemaphoreType.DMA,
        pltpu.SemaphoreType.DMA,
        pltpu.SemaphoreType.DMA,
        pltpu.SemaphoreType.DMA,
    ],
)(_agg_body)


def _fin_body(acc_ref, h2_ref, cnt_ref, b_ref, o_ref):
    cnt = jnp.sum(cnt_ref[...], axis=0)
    dinv = lax.rsqrt(cnt + 1.0)[:, None]
    s = acc_ref[0] + acc_ref[1] + h2_ref[...]
    o_ref[...] = jnp.maximum(s * dinv + b_ref[...], 0.0)


def kernel(x, edge_index, W, b):
    N = x.shape[0]
    E = edge_index.shape[1]
    ei = edge_index.astype(jnp.int32)
    pad = jnp.full((EP - E,), NP - 1, jnp.int32)
    src = jnp.concatenate([ei[0], pad])
    dst = jnp.concatenate([ei[1], pad])
    srcg = src.reshape(NW, GPT, CH)
    srcg = jnp.concatenate([srcg, jnp.zeros((NW, 2, CH), jnp.int32)], axis=1)
    dstg = dst.reshape(NW, NBLK, BLK, SCH)
    dstg = jnp.concatenate(
        [dstg, jnp.zeros((NW, 2, BLK, SCH), jnp.int32)], axis=1)
    dst_flat = dst.reshape(NW, EPT)
    x_p = jnp.pad(x, ((0, NP - N), (0, 0)))

    cnt_part = _count_kernel(dst_flat)

    h2 = pl.pallas_call(
        _mm_body,
        grid=(GRID,),
        in_specs=[
            pl.BlockSpec((ROWB, D), lambda i: (i, 0)),
            pl.BlockSpec((D, D), lambda i: (0, 0)),
            pl.BlockSpec((NW, ROWB), lambda i: (0, i)),
        ],
        out_specs=pl.BlockSpec((ROWB, D), lambda i: (i, 0)),
        out_shape=jax.ShapeDtypeStruct((NP, D), jnp.float32),
    )(x_p, W, cnt_part)

    accp = _agg_kernel(h2, srcg, dstg)

    out_p = pl.pallas_call(
        _fin_body,
        grid=(GRID,),
        in_specs=[
            pl.BlockSpec((NC, ROWB, D), lambda i: (0, i, 0)),
            pl.BlockSpec((ROWB, D), lambda i: (i, 0)),
            pl.BlockSpec((NW, ROWB), lambda i: (0, i)),
            pl.BlockSpec((1, D), lambda i: (0, 0)),
        ],
        out_specs=pl.BlockSpec((ROWB, D), lambda i: (i, 0)),
        out_shape=jax.ShapeDtypeStruct((NP, D), jnp.float32),
    )(accp, h2, cnt_part, b.reshape(1, D))

    return out_p[:N]
